# baseline (speedup 1.0000x reference)
import functools

import jax
import jax.numpy as jnp
from jax import lax
from jax.experimental import pallas as pl
from jax.experimental.pallas import tpu as pltpu
from jax.experimental.pallas import tpu_sc as plsc

H, S, D, K = 16, 2048, 128, 64
SP = S // 2
QB = 256
NC, NS = 2, 16
NW = NC * NS
WPW = H * SP // NW
CW = 16
ITERS = WPW // CW


def _tc_scores_body(ql_ref, qh_ref, k_ref, out_ref):
    dn = (((1,), (1,)), ((), ()))
    s_lo = lax.dot_general(ql_ref[0], k_ref[0], dn,
                           preferred_element_type=jnp.float32)
    s_hi = lax.dot_general(qh_ref[0], k_ref[0], dn,
                           preferred_element_type=jnp.float32)
    st = jnp.stack([s_lo.astype(jnp.bfloat16), s_hi.astype(jnp.bfloat16)],
                   axis=1).reshape(2 * QB, S)
    out_ref[0] = pltpu.bitcast(st, jnp.int32)


def _tc_scores(qh, kh, g, nh):
    nj = SP // QB
    h0 = g * nh
    return pl.pallas_call(
        _tc_scores_body,
        grid=(nh, nj),
        in_specs=[
            pl.BlockSpec((1, QB, D), lambda h, j, h0=h0: (h + h0, j, 0)),
            pl.BlockSpec((1, QB, D),
                         lambda h, j, h0=h0, nj=nj: (h + h0, j + nj, 0)),
            pl.BlockSpec((1, S, D), lambda h, j, h0=h0: (h + h0, 0, 0)),
        ],
        out_specs=pl.BlockSpec((1, QB, S), lambda h, j: (h, j, 0)),
        out_shape=jax.ShapeDtypeStruct((nh, SP, S), jnp.int32),
    )(qh, qh, kh)


def _sc_gather_body(nh, packed_hbm, idx_hbm, out_hbm, wbuf, idxv, outv,
                    sem_w, sem_i, sem_o):
    wpw = nh * SP // NW
    iters = wpw // CW
    parts = NW // nh
    w = lax.axis_index("c") * NS + lax.axis_index("s")
    h = w // parts
    part = w % parts
    wr_base = h * SP + part * wpw
    s_lo_base = h * S + part * wpw
    s_hi_base = s_lo_base + SP

    def offsets(c):
        off = c * CW
        return (pl.multiple_of(wr_base + off, CW),
                pl.multiple_of(s_lo_base + off, CW),
                pl.multiple_of(s_hi_base + off, CW))

    def start_in(c, b):
        wr0, sl0, sh0 = offsets(c)
        pltpu.async_copy(packed_hbm.at[pl.ds(wr0, CW), :], wbuf.at[b],
                         sem_w[b])
        pltpu.async_copy(idx_hbm.at[pl.ds(sl0, CW), :],
                         idxv.at[b, pl.ds(0, CW)], sem_i[b])
        pltpu.async_copy(idx_hbm.at[pl.ds(sh0, CW), :],
                         idxv.at[b, pl.ds(CW, CW)], sem_i[b])

    def wait_in(b):
        pltpu.make_async_copy(packed_hbm.at[pl.ds(0, CW), :], wbuf.at[b],
                              sem_w[b]).wait()
        pltpu.make_async_copy(idx_hbm.at[pl.ds(0, CW), :],
                              idxv.at[b, pl.ds(0, CW)], sem_i[b]).wait()
        pltpu.make_async_copy(idx_hbm.at[pl.ds(0, CW), :],
                              idxv.at[b, pl.ds(CW, CW)], sem_i[b]).wait()

    def start_out(c, b):
        _, sl0, sh0 = offsets(c)
        pltpu.async_copy(outv.at[b, pl.ds(0, CW)],
                         out_hbm.at[pl.ds(sl0, CW), :], sem_o[b])
        pltpu.async_copy(outv.at[b, pl.ds(CW, CW)],
                         out_hbm.at[pl.ds(sh0, CW), :], sem_o[b])

    def wait_out(b):
        pltpu.make_async_copy(outv.at[b, pl.ds(0, CW)],
                              out_hbm.at[pl.ds(0, CW), :], sem_o[b]).wait()
        pltpu.make_async_copy(outv.at[b, pl.ds(CW, CW)],
                              out_hbm.at[pl.ds(CW, CW), :], sem_o[b]).wait()

    start_in(0, 0)
    start_in(1, 1)

    def body(i2, carry):
        ci = i2 * 2
        for b in (0, 1):
            c = ci + b
            wait_in(b)

            @pl.when(c >= 2)
            def _():
                wait_out(b)

            for r in range(CW):
                row = jnp.full((16,), r, jnp.int32)
                for jg in range(K // 16):
                    col = idxv[b, r, pl.ds(jg * 16, 16)]
                    wv = plsc.load_gather(wbuf.at[b], [row, col])
                    outv[b, r, pl.ds(jg * 16, 16)] = wv
                    col2 = idxv[b, CW + r, pl.ds(jg * 16, 16)]
                    wv2 = plsc.load_gather(wbuf.at[b], [row, col2])
                    outv[b, CW + r, pl.ds(jg * 16, 16)] = (
                        lax.shift_right_logical(wv2, jnp.int32(16)))
            start_out(c, b)

            @pl.when(c + 2 < iters)
            def _():
                start_in(c + 2, b)
        return carry

    lax.fori_loop(0, iters // 2, body, 0)
    wait_out(0)
    wait_out(1)


@functools.cache
def _sc_gather_kernel(nh):
    return pl.kernel(
        functools.partial(_sc_gather_body, nh),
        out_type=jax.ShapeDtypeStruct((nh * S, K), jnp.int32),
        mesh=plsc.VectorSubcoreMesh(core_axis_name="c", subcore_axis_name="s",
                                    num_cores=NC, num_subcores=NS),
        scratch_types=[
            pltpu.VMEM((2, CW, S), jnp.int32),
            pltpu.VMEM((2, 2 * CW, K), jnp.int32),
            pltpu.VMEM((2, 2 * CW, K), jnp.int32),
            [pltpu.SemaphoreType.DMA, pltpu.SemaphoreType.DMA],
            [pltpu.SemaphoreType.DMA, pltpu.SemaphoreType.DMA],
            [pltpu.SemaphoreType.DMA, pltpu.SemaphoreType.DMA],
        ],
        compiler_params=pltpu.CompilerParams(needs_layout_passes=False),
    )


NH = 4


def kernel(q, k, idx):
    qh, kh, ih = q[0], k[0], idx[0]
    idx_flat = ih.reshape(H * S, K)
    parts = []
    for g in range(H // NH):
        packed = _tc_scores(qh, kh, g, NH)
        bits = _sc_gather_kernel(NH)(
            packed.reshape(NH * SP, S),
            lax.slice_in_dim(idx_flat, g * NH * S, (g + 1) * NH * S))
        out_g = lax.bitcast_convert_type(bits.astype(jnp.uint16),
                                         jnp.bfloat16)
        parts.append(out_g.reshape(1, NH, S, K))
    out = (jnp.concatenate(parts, axis=1) if len(parts) > 1 else parts[0])
    return out


# speedup vs baseline: 2.0787x; 2.0787x over previous
import functools

import jax
import jax.numpy as jnp
from jax import lax
from jax.experimental import pallas as pl
from jax.experimental.pallas import tpu as pltpu
from jax.experimental.pallas import tpu_sc as plsc

H, S, D, K = 16, 2048, 128, 64
SP = S // 2
QB = 512
NC, NS = 2, 16
NW = NC * NS
WPW = H * SP // NW
CW = 16
ITERS = WPW // CW


def _tc_scores_body(ql_ref, qh_ref, k_ref, out_ref):
    dn = (((1,), (1,)), ((), ()))
    s_lo = lax.dot_general(ql_ref[0], k_ref[0], dn,
                           preferred_element_type=jnp.float32)
    s_hi = lax.dot_general(qh_ref[0], k_ref[0], dn,
                           preferred_element_type=jnp.float32)
    lo = lax.convert_element_type(
        lax.bitcast_convert_type(s_lo.astype(jnp.bfloat16), jnp.uint16),
        jnp.uint32)
    hi = lax.convert_element_type(
        lax.bitcast_convert_type(s_hi.astype(jnp.bfloat16), jnp.uint16),
        jnp.uint32)
    out_ref[0] = lax.bitcast_convert_type(lo | (hi << 16), jnp.int32)


def _tc_scores(qh, kh, g, nh):
    nj = SP // QB
    h0 = g * nh
    return pl.pallas_call(
        _tc_scores_body,
        grid=(nh, nj),
        in_specs=[
            pl.BlockSpec((1, QB, D), lambda h, j, h0=h0: (h + h0, j, 0)),
            pl.BlockSpec((1, QB, D),
                         lambda h, j, h0=h0, nj=nj: (h + h0, j + nj, 0)),
            pl.BlockSpec((1, S, D), lambda h, j, h0=h0: (h + h0, 0, 0)),
        ],
        out_specs=pl.BlockSpec((1, QB, S), lambda h, j: (h, j, 0)),
        out_shape=jax.ShapeDtypeStruct((nh, SP, S), jnp.int32),
    )(qh, qh, kh)


def _sc_gather_body(nh, packed_hbm, idx_hbm, out_hbm, wbuf, idxv, outv,
                    sem_w, sem_i, sem_o):
    wpw = nh * SP // NW
    iters = wpw // CW
    parts = NW // nh
    w = lax.axis_index("c") * NS + lax.axis_index("s")
    h = w // parts
    part = w % parts
    wr_base = h * SP + part * wpw
    s_lo_base = h * S + part * wpw
    s_hi_base = s_lo_base + SP

    def offsets(c):
        off = c * CW
        return (pl.multiple_of(wr_base + off, CW),
                pl.multiple_of(s_lo_base + off, CW),
                pl.multiple_of(s_hi_base + off, CW))

    def start_in(c, b):
        wr0, sl0, sh0 = offsets(c)
        pltpu.async_copy(packed_hbm.at[pl.ds(wr0, CW), :], wbuf.at[b],
                         sem_w[b])
        pltpu.async_copy(idx_hbm.at[pl.ds(sl0, CW), :],
                         idxv.at[b, pl.ds(0, CW)], sem_i[b])
        pltpu.async_copy(idx_hbm.at[pl.ds(sh0, CW), :],
                         idxv.at[b, pl.ds(CW, CW)], sem_i[b])

    def wait_in(b):
        pltpu.make_async_copy(packed_hbm.at[pl.ds(0, CW), :], wbuf.at[b],
                              sem_w[b]).wait()
        pltpu.make_async_copy(idx_hbm.at[pl.ds(0, CW), :],
                              idxv.at[b, pl.ds(0, CW)], sem_i[b]).wait()
        pltpu.make_async_copy(idx_hbm.at[pl.ds(0, CW), :],
                              idxv.at[b, pl.ds(CW, CW)], sem_i[b]).wait()

    def start_out(c, b):
        _, sl0, sh0 = offsets(c)
        pltpu.async_copy(outv.at[b, pl.ds(0, CW)],
                         out_hbm.at[pl.ds(sl0, CW), :], sem_o[b])
        pltpu.async_copy(outv.at[b, pl.ds(CW, CW)],
                         out_hbm.at[pl.ds(sh0, CW), :], sem_o[b])

    def wait_out(b):
        pltpu.make_async_copy(outv.at[b, pl.ds(0, CW)],
                              out_hbm.at[pl.ds(0, CW), :], sem_o[b]).wait()
        pltpu.make_async_copy(outv.at[b, pl.ds(CW, CW)],
                              out_hbm.at[pl.ds(CW, CW), :], sem_o[b]).wait()

    start_in(0, 0)
    start_in(1, 1)

    def body(i2, carry):
        ci = i2 * 2
        for b in (0, 1):
            c = ci + b
            wait_in(b)

            @pl.when(c >= 2)
            def _():
                wait_out(b)

            for r in range(CW):
                row = jnp.full((16,), r, jnp.int32)
                for jg in range(K // 16):
                    col = idxv[b, r, pl.ds(jg * 16, 16)]
                    wv = plsc.load_gather(wbuf.at[b], [row, col])
                    outv[b, r, pl.ds(jg * 16, 16)] = wv
                    col2 = idxv[b, CW + r, pl.ds(jg * 16, 16)]
                    wv2 = plsc.load_gather(wbuf.at[b], [row, col2])
                    outv[b, CW + r, pl.ds(jg * 16, 16)] = (
                        lax.shift_right_logical(wv2, jnp.int32(16)))
            start_out(c, b)

            @pl.when(c + 2 < iters)
            def _():
                start_in(c + 2, b)
        return carry

    lax.fori_loop(0, iters // 2, body, 0)
    wait_out(0)
    wait_out(1)


@functools.cache
def _sc_gather_kernel(nh):
    return pl.kernel(
        functools.partial(_sc_gather_body, nh),
        out_type=jax.ShapeDtypeStruct((nh * S, K), jnp.int32),
        mesh=plsc.VectorSubcoreMesh(core_axis_name="c", subcore_axis_name="s",
                                    num_cores=NC, num_subcores=NS),
        scratch_types=[
            pltpu.VMEM((2, CW, S), jnp.int32),
            pltpu.VMEM((2, 2 * CW, K), jnp.int32),
            pltpu.VMEM((2, 2 * CW, K), jnp.int32),
            [pltpu.SemaphoreType.DMA, pltpu.SemaphoreType.DMA],
            [pltpu.SemaphoreType.DMA, pltpu.SemaphoreType.DMA],
            [pltpu.SemaphoreType.DMA, pltpu.SemaphoreType.DMA],
        ],
        compiler_params=pltpu.CompilerParams(needs_layout_passes=False),
    )


NH = 4


def kernel(q, k, idx):
    qh, kh, ih = q[0], k[0], idx[0]
    idx_flat = ih.reshape(H * S, K)
    parts = []
    for g in range(H // NH):
        packed = _tc_scores(qh, kh, g, NH)
        bits = _sc_gather_kernel(NH)(
            packed.reshape(NH * SP, S),
            lax.slice_in_dim(idx_flat, g * NH * S, (g + 1) * NH * S))
        out_g = lax.bitcast_convert_type(bits.astype(jnp.uint16),
                                         jnp.bfloat16)
        parts.append(out_g.reshape(1, NH, S, K))
    out = (jnp.concatenate(parts, axis=1) if len(parts) > 1 else parts[0])
    return out
